# Initial kernel scaffold; baseline (speedup 1.0000x reference)
import jax
import jax.numpy as jnp
from jax.experimental import pallas as pl


def kernel(x, W_tgt, W_pos):
    raise NotImplementedError("write your pallas kernel here")


# speedup vs baseline: 1.7281x; 1.7281x over previous
import jax
import jax.numpy as jnp
from jax import lax
from jax.experimental import pallas as pl
from jax.experimental.pallas import tpu as pltpu
from jax.experimental.pallas import tpu_sc as plsc

N_CORES = 2
N_SUBCORES = 16
NW = N_CORES * N_SUBCORES
CHUNK = 100


def _body(x_hbm, tgt_hbm, pos_hbm, out_hbm, idx_v, rows_v, pos_v, sem):
    wid = lax.axis_index("s") * N_CORES + lax.axis_index("c")
    nchunks = x_hbm.shape[0]
    nper = nchunks // NW
    pltpu.sync_copy(pos_hbm, pos_v)

    def iter_fn(k, carry):
        i = wid * nper + k
        pltpu.sync_copy(x_hbm.at[i], idx_v)
        pltpu.async_copy(tgt_hbm.at[idx_v], rows_v, sem).wait()
        half = lax.rem(i, 2) * CHUNK

        def add_fn(r, c):
            pr = half + r
            for j in range(4):
                sl = pl.ds(j * 16, 16)
                rows_v[r, sl] = rows_v[r, sl] + pos_v[pr, sl]
            return c

        lax.fori_loop(0, CHUNK, add_fn, 0)
        pltpu.sync_copy(rows_v, out_hbm.at[i])
        return carry

    lax.fori_loop(0, nper, iter_fn, 0)


def kernel(x, W_tgt, W_pos):
    B, S = x.shape
    D = W_tgt.shape[1]
    nchunks = (B * S) // CHUNK
    x2 = x.reshape(nchunks, CHUNK).astype(jnp.int32)
    mesh = plsc.VectorSubcoreMesh(core_axis_name="c", subcore_axis_name="s")
    f = pl.kernel(
        _body,
        mesh=mesh,
        compiler_params=pltpu.CompilerParams(use_tc_tiling_on_sc=False),
        out_type=jax.ShapeDtypeStruct((nchunks, CHUNK, D), jnp.float32),
        scratch_types=[
            pltpu.VMEM((CHUNK,), jnp.int32),
            pltpu.VMEM((CHUNK, D), jnp.float32),
            pltpu.VMEM((S, D), jnp.float32),
            pltpu.SemaphoreType.DMA,
        ],
    )
    out = f(x2, W_tgt, W_pos)
    return out.reshape(B, S, D)


# speedup vs baseline: 2.6322x; 1.5232x over previous
import jax
import jax.numpy as jnp
from jax import lax
from jax.experimental import pallas as pl
from jax.experimental.pallas import tpu as pltpu
from jax.experimental.pallas import tpu_sc as plsc

N_CORES = 2
N_SUBCORES = 16
NW = N_CORES * N_SUBCORES
CHUNK = 100


def _body(x_hbm, tgt_hbm, pos_hbm, out_hbm, pos_sh, idx0, idx1, rows0, rows1,
          gsem0, gsem1, osem0, osem1):
    cid = lax.axis_index("c")
    sid = lax.axis_index("s")
    wid = sid * N_CORES + cid
    nper = out_hbm.shape[0] // NW
    base = wid * nper

    @pl.when(sid == 0)
    def _():
        pltpu.sync_copy(pos_hbm, pos_sh)

    plsc.subcore_barrier()

    def fire(i, idxb, rowsb, gsemb):
        pltpu.sync_copy(pos_sh, rowsb)
        pltpu.sync_copy(x_hbm.at[i], idxb)
        pltpu.async_copy(tgt_hbm.at[idxb.at[0]], rowsb.at[pl.ds(0, CHUNK)],
                         gsemb, add=True)
        pltpu.async_copy(tgt_hbm.at[idxb.at[1]], rowsb.at[pl.ds(CHUNK, CHUNK)],
                         gsemb, add=True)

    def wait_gather(idxb, rowsb, gsemb):
        pltpu.make_async_copy(tgt_hbm.at[idxb.at[0]],
                              rowsb.at[pl.ds(0, CHUNK)], gsemb).wait()
        pltpu.make_async_copy(tgt_hbm.at[idxb.at[1]],
                              rowsb.at[pl.ds(CHUNK, CHUNK)], gsemb).wait()

    def wait_out(rowsb, osemb, i):
        pltpu.make_async_copy(rowsb, out_hbm.at[i], osemb).wait()

    def pair(p, carry):
        i0 = base + 2 * p
        i1 = i0 + 1

        @pl.when(p >= 1)
        def _():
            wait_out(rows0, osem0, i0)

        fire(i0, idx0, rows0, gsem0)

        @pl.when(p >= 1)
        def _():
            wait_gather(idx1, rows1, gsem1)
            pltpu.async_copy(rows1, out_hbm.at[i0 - 1], osem1)

        @pl.when(p >= 1)
        def _():
            wait_out(rows1, osem1, i1)

        fire(i1, idx1, rows1, gsem1)
        wait_gather(idx0, rows0, gsem0)
        pltpu.async_copy(rows0, out_hbm.at[i0], osem0)
        return carry

    lax.fori_loop(0, nper // 2, pair, 0)

    last = base + nper - 1
    wait_gather(idx1, rows1, gsem1)
    pltpu.async_copy(rows1, out_hbm.at[last], osem1)
    wait_out(rows0, osem0, last - 1)
    wait_out(rows1, osem1, last)


def kernel(x, W_tgt, W_pos):
    B, S = x.shape
    D = W_tgt.shape[1]
    x3 = x.reshape(B, S // CHUNK, CHUNK).astype(jnp.int32)
    mesh = plsc.VectorSubcoreMesh(core_axis_name="c", subcore_axis_name="s")
    f = pl.kernel(
        _body,
        mesh=mesh,
        compiler_params=pltpu.CompilerParams(use_tc_tiling_on_sc=False),
        out_type=jax.ShapeDtypeStruct((B, S, D), jnp.float32),
        scratch_types=[
            pltpu.VMEM_SHARED((S, D), jnp.float32),
            pltpu.VMEM((S // CHUNK, CHUNK), jnp.int32),
            pltpu.VMEM((S // CHUNK, CHUNK), jnp.int32),
            pltpu.VMEM((S, D), jnp.float32),
            pltpu.VMEM((S, D), jnp.float32),
            pltpu.SemaphoreType.DMA,
            pltpu.SemaphoreType.DMA,
            pltpu.SemaphoreType.DMA,
            pltpu.SemaphoreType.DMA,
        ],
    )
    return f(x3, W_tgt, W_pos)
